# baseline (speedup 1.0000x reference)
import functools

import jax
import jax.numpy as jnp
from jax import lax
from jax.experimental import pallas as pl
from jax.experimental.pallas import tpu as pltpu
from jax.experimental.pallas import tpu_sc as plsc

_NC = 2
_NS = 16
_NW = _NC * _NS
_L = 16


def _pooled_sum(emb_table, text_t):
    hist, batch = text_t.shape
    vocab, dim = emb_table.shape
    b_per_w = batch // _NW
    n_slice = dim // _L

    mesh = plsc.VectorSubcoreMesh(
        core_axis_name="c", subcore_axis_name="s",
        num_cores=_NC, num_subcores=_NS)

    @functools.partial(
        pl.kernel,
        out_type=jax.ShapeDtypeStruct((batch, dim), jnp.float32),
        mesh=mesh,
        scratch_types=[
            pltpu.VMEM((hist, b_per_w), jnp.int32),
            pltpu.VMEM((b_per_w, dim), jnp.float32),
            pltpu.SemaphoreType.DMA,
            pltpu.SemaphoreType.DMA,
        ],
    )
    def k(table_hbm, textt_hbm, out_hbm, idx_v, acc_v, sem, idx_sem):
        wid = lax.axis_index("s") * _NC + lax.axis_index("c")
        base = wid * b_per_w
        idx_cp = pltpu.async_copy(
            textt_hbm.at[:, pl.ds(base, b_per_w)], idx_v, idx_sem)

        zero = jnp.zeros((_L,), jnp.float32)

        def zloop(r, carry):
            for s in range(n_slice):
                acc_v[r, pl.ds(s * _L, _L)] = zero
            return carry

        lax.fori_loop(0, b_per_w, zloop, 0)
        idx_cp.wait()

        def fire(j, carry):
            pltpu.async_copy(table_hbm.at[idx_v.at[j]], acc_v, sem, add=True)
            return carry

        lax.fori_loop(0, hist, fire, 0)

        def drain(j, carry):
            pltpu.make_async_copy(table_hbm.at[idx_v.at[0]], acc_v, sem).wait()
            return carry

        lax.fori_loop(0, hist, drain, 0)
        pltpu.sync_copy(acc_v, out_hbm.at[pl.ds(base, b_per_w)])

    return k(emb_table, text_t)


def _head_body(x_ref, w_ref, b_ref, o_ref, *, inv_len):
    x = jnp.tanh(x_ref[...] * inv_len)
    logits = lax.dot_general(
        w_ref[...], x, dimension_numbers=(((1,), (1,)), ((), ())),
        preferred_element_type=jnp.float32)
    logits = logits + b_ref[...]
    m = jnp.max(logits, axis=0, keepdims=True)
    e = jnp.exp(logits - m)
    o_ref[...] = e / jnp.sum(e, axis=0, keepdims=True)


def kernel(text, offsets, emb_table, fc_w, fc_b):
    del offsets
    batch, hist = text.shape
    num_class = fc_w.shape[0]
    text_t = text.astype(jnp.int32).T
    pooled = _pooled_sum(emb_table, text_t)
    head = pl.pallas_call(
        functools.partial(_head_body, inv_len=1.0 / hist),
        out_shape=jax.ShapeDtypeStruct((num_class, batch), jnp.float32),
    )
    return head(pooled, fc_w, fc_b.reshape(num_class, 1)).T
